# baseline (speedup 1.0000x reference)
import functools

import jax
import jax.numpy as jnp
from jax import lax
from jax.experimental import pallas as pl
from jax.experimental.pallas import tpu as pltpu

B, S, T, M = 32, 200, 1000, 80
N_SPK, N_EMO = 256, 8

ROWS = B * T
BLK = 4000
NC = ROWS // BLK


def _ce_from_refs(logits, labels):
    mx = jnp.max(logits, axis=-1, keepdims=True)
    lse = mx + jnp.log(jnp.sum(jnp.exp(logits - mx), axis=-1, keepdims=True))
    logp = logits - lse
    onehot = lax.broadcasted_iota(jnp.int32, logits.shape, 1) == labels
    picked = jnp.sum(jnp.where(onehot, logp, 0.0))
    return -picked / float(B)


def _body(mel_ref, out_ref, post_ref, mv_ref,
          mvd_ref, pp_ref, pt_ref, ep_ref, et_ref,
          dp_ref, dt_ref, sv_ref,
          s1_ref, s2_ref, e1_ref, e2_ref, spk_ref, emo_ref,
          o_ref, acc_ref):
    i = pl.program_id(0)

    mv = mv_ref[...]
    mel = mel_ref[...]
    cm = jnp.sum(jnp.abs(out_ref[...] - mel) * mv)
    cp = jnp.sum(jnp.abs(post_ref[...] - mel) * mv)
    cs = jnp.sum(mv)

    @pl.when(i == 0)
    def _():
        acc_ref[0] = cm
        acc_ref[1] = cp
        acc_ref[2] = cs

    @pl.when(i > 0)
    def _():
        acc_ref[0] += cm
        acc_ref[1] += cp
        acc_ref[2] += cs

    @pl.when(i == NC - 1)
    def _():
        s_mask = acc_ref[2]
        denom3 = jnp.maximum(s_mask * float(M), 1.0)
        mel_loss = acc_ref[0] / denom3
        postnet_mel_loss = acc_ref[1] / denom3

        mvd = mvd_ref[...]
        denom1 = jnp.maximum(s_mask, 1.0)
        pitch_loss = jnp.sum((pp_ref[...] - pt_ref[...]) ** 2 * mvd) / denom1
        energy_loss = jnp.sum((ep_ref[...] - et_ref[...]) ** 2 * mvd) / denom1

        sv = sv_ref[...]
        log_dur = jnp.log(dt_ref[...] + 1.0)
        duration_loss = (jnp.sum((dp_ref[...] - log_dur) ** 2 * sv)
                         / jnp.maximum(jnp.sum(sv), 1.0))

        spk = spk_ref[...]
        emo = emo_ref[...]
        speaker_loss_1 = _ce_from_refs(s1_ref[...], spk)
        speaker_loss_2 = _ce_from_refs(s2_ref[...], spk)
        emotion_loss_1 = _ce_from_refs(e1_ref[...], emo)
        emotion_loss_2 = _ce_from_refs(e2_ref[...], emo)

        all_loss = (mel_loss + postnet_mel_loss + pitch_loss + energy_loss
                    + duration_loss)
        total_loss = (all_loss + speaker_loss_1 + emotion_loss_1
                      + speaker_loss_2 + emotion_loss_2)

        vals = (mel_loss, postnet_mel_loss, pitch_loss, energy_loss,
                duration_loss, speaker_loss_1, speaker_loss_2,
                emotion_loss_1, emotion_loss_2, total_loss)
        col = lax.broadcasted_iota(jnp.int32, (8, 128), 1)
        row = lax.broadcasted_iota(jnp.int32, (8, 128), 0)
        acc = jnp.zeros((8, 128), jnp.float32)
        for k, v in enumerate(vals):
            acc = jnp.where((row == 0) & (col == k), v, acc)
        o_ref[...] = acc


@jax.jit
def _run(mels, pitches, energies, durations, speakers, emotions, output,
         postnet_output, p_preds, e_preds, d_preds, src_masks, mel_masks,
         spk_cls_1_output, spk_cls_2_output, emo_cls_1_output,
         emo_cls_2_output):
    mel_valid = (~mel_masks).astype(jnp.float32)
    src_valid = (~src_masks).astype(jnp.float32)

    mel2 = mels.reshape(ROWS, M)
    out2 = output.reshape(ROWS, M)
    post2 = postnet_output.reshape(ROWS, M)
    mv_col = mel_valid.reshape(ROWS, 1)

    dense = (B * T) // 128
    mv_dense = mel_valid.reshape(dense, 128)
    ppd = p_preds.reshape(dense, 128)
    ptd = pitches.reshape(dense, 128)
    epd = e_preds.reshape(dense, 128)
    etd = energies.reshape(dense, 128)

    dur_f = durations.astype(jnp.float32)
    spk = speakers.astype(jnp.int32).reshape(B, 1)
    emo = emotions.astype(jnp.int32).reshape(B, 1)

    chunk = lambda r, c: pl.BlockSpec((r, c), lambda i: (i, 0))
    whole = lambda r, c: pl.BlockSpec((r, c), lambda i: (0, 0))

    out = pl.pallas_call(
        _body,
        grid=(NC,),
        in_specs=[
            chunk(BLK, M), chunk(BLK, M), chunk(BLK, M), chunk(BLK, 1),
            whole(dense, 128), whole(dense, 128), whole(dense, 128),
            whole(dense, 128), whole(dense, 128),
            whole(B, S), whole(B, S), whole(B, S),
            whole(B, N_SPK), whole(B, N_SPK),
            whole(B, N_EMO), whole(B, N_EMO),
            whole(B, 1), whole(B, 1),
        ],
        out_specs=pl.BlockSpec((8, 128), lambda i: (0, 0)),
        out_shape=jax.ShapeDtypeStruct((8, 128), jnp.float32),
        scratch_shapes=[pltpu.SMEM((4,), jnp.float32)],
    )(mel2, out2, post2, mv_col,
      mv_dense, ppd, ptd, epd, etd,
      d_preds, dur_f, src_valid,
      spk_cls_1_output, spk_cls_2_output,
      emo_cls_1_output, emo_cls_2_output,
      spk, emo)
    return tuple(out[0, k] for k in range(10))


def kernel(mels, pitches, energies, durations, speakers, emotions, output,
           postnet_output, p_preds, e_preds, d_preds, src_masks, mel_masks,
           spk_cls_1_output, spk_cls_2_output, emo_cls_1_output,
           emo_cls_2_output):
    return _run(mels, pitches, energies, durations, speakers, emotions,
                output, postnet_output, p_preds, e_preds, d_preds,
                src_masks, mel_masks, spk_cls_1_output, spk_cls_2_output,
                emo_cls_1_output, emo_cls_2_output)


# speedup vs baseline: 1.0269x; 1.0269x over previous
import jax
import jax.numpy as jnp
from jax import lax
from jax.experimental import pallas as pl
from jax.experimental.pallas import tpu as pltpu

B, S, T, M = 32, 200, 1000, 80
N_SPK, N_EMO = 256, 8

ROWS = B * T
BLK = 4000
NC = ROWS // BLK

_DOT = (((1,), (0,)), ((), ()))


def _ce(logits, labels):
    mx = jnp.max(logits, axis=-1, keepdims=True)
    lse = mx + jnp.log(jnp.sum(jnp.exp(logits - mx), axis=-1, keepdims=True))
    logp = logits - lse
    onehot = lax.broadcasted_iota(jnp.int32, logits.shape, 1) == labels
    picked = jnp.sum(jnp.where(onehot, logp, 0.0))
    return -picked / float(B)


def _body(mel_ref, out_ref, post_ref, mv_ref,
          mvd_ref, pp_ref, pt_ref, ep_ref, et_ref,
          dp_ref, dt_ref, sv_ref,
          s1_ref, s2_ref, e1_ref, e2_ref, spk_ref, emo_ref,
          o_ref, acc_ref):
    i = pl.program_id(0)

    mv = mv_ref[0]
    mel = mel_ref[...]
    dm = jnp.abs(out_ref[...] - mel)
    dp = jnp.abs(post_ref[...] - mel)
    cm = lax.dot_general(mv, dm, _DOT, preferred_element_type=jnp.float32)
    cp = lax.dot_general(mv, dp, _DOT, preferred_element_type=jnp.float32)

    @pl.when(i == 0)
    def _():
        acc_ref[0:1, :] = cm
        acc_ref[1:2, :] = cp

    @pl.when(i > 0)
    def _():
        acc_ref[0:1, :] += cm
        acc_ref[1:2, :] += cp

    @pl.when(i == NC - 1)
    def _():
        mvd = mvd_ref[...]
        s_mask = jnp.sum(mvd)
        denom3 = jnp.maximum(s_mask * float(M), 1.0)
        mel_loss = jnp.sum(acc_ref[0:1, :]) / denom3
        postnet_mel_loss = jnp.sum(acc_ref[1:2, :]) / denom3

        denom1 = jnp.maximum(s_mask, 1.0)
        pitch_loss = jnp.sum((pp_ref[...] - pt_ref[...]) ** 2 * mvd) / denom1
        energy_loss = jnp.sum((ep_ref[...] - et_ref[...]) ** 2 * mvd) / denom1

        sv = sv_ref[...]
        log_dur = jnp.log(dt_ref[...] + 1.0)
        duration_loss = (jnp.sum((dp_ref[...] - log_dur) ** 2 * sv)
                         / jnp.maximum(jnp.sum(sv), 1.0))

        spk = spk_ref[...]
        emo = emo_ref[...]
        speaker_loss_1 = _ce(s1_ref[...], spk)
        speaker_loss_2 = _ce(s2_ref[...], spk)
        emotion_loss_1 = _ce(e1_ref[...], emo)
        emotion_loss_2 = _ce(e2_ref[...], emo)

        all_loss = (mel_loss + postnet_mel_loss + pitch_loss + energy_loss
                    + duration_loss)
        total_loss = (all_loss + speaker_loss_1 + emotion_loss_1
                      + speaker_loss_2 + emotion_loss_2)

        vals = (mel_loss, postnet_mel_loss, pitch_loss, energy_loss,
                duration_loss, speaker_loss_1, speaker_loss_2,
                emotion_loss_1, emotion_loss_2, total_loss)
        col = lax.broadcasted_iota(jnp.int32, (8, 128), 1)
        row = lax.broadcasted_iota(jnp.int32, (8, 128), 0)
        res = jnp.zeros((8, 128), jnp.float32)
        for k, v in enumerate(vals):
            res = jnp.where((row == 0) & (col == k), v, res)
        o_ref[...] = res


@jax.jit
def _run(mels, pitches, energies, durations, speakers, emotions, output,
         postnet_output, p_preds, e_preds, d_preds, src_masks, mel_masks,
         spk_cls_1_output, spk_cls_2_output, emo_cls_1_output,
         emo_cls_2_output):
    mel_valid = (~mel_masks).astype(jnp.float32)
    src_valid = (~src_masks).astype(jnp.float32)

    mel2 = mels.reshape(ROWS, M)
    out2 = output.reshape(ROWS, M)
    post2 = postnet_output.reshape(ROWS, M)
    mv_chunks = mel_valid.reshape(NC, 1, BLK)

    dense = ROWS // 128
    mv_dense = mel_valid.reshape(dense, 128)
    ppd = p_preds.reshape(dense, 128)
    ptd = pitches.reshape(dense, 128)
    epd = e_preds.reshape(dense, 128)
    etd = energies.reshape(dense, 128)

    dur_f = durations.astype(jnp.float32)
    spk = speakers.astype(jnp.int32).reshape(B, 1)
    emo = emotions.astype(jnp.int32).reshape(B, 1)

    chunk = lambda r, c: pl.BlockSpec((r, c), lambda i: (i, 0))
    whole = lambda r, c: pl.BlockSpec((r, c), lambda i: (0, 0))

    out = pl.pallas_call(
        _body,
        grid=(NC,),
        in_specs=[
            chunk(BLK, M), chunk(BLK, M), chunk(BLK, M),
            pl.BlockSpec((1, 1, BLK), lambda i: (i, 0, 0)),
            whole(dense, 128), whole(dense, 128), whole(dense, 128),
            whole(dense, 128), whole(dense, 128),
            whole(B, S), whole(B, S), whole(B, S),
            whole(B, N_SPK), whole(B, N_SPK),
            whole(B, N_EMO), whole(B, N_EMO),
            whole(B, 1), whole(B, 1),
        ],
        out_specs=pl.BlockSpec((8, 128), lambda i: (0, 0)),
        out_shape=jax.ShapeDtypeStruct((8, 128), jnp.float32),
        scratch_shapes=[pltpu.VMEM((8, M), jnp.float32)],
    )(mel2, out2, post2, mv_chunks,
      mv_dense, ppd, ptd, epd, etd,
      d_preds, dur_f, src_valid,
      spk_cls_1_output, spk_cls_2_output,
      emo_cls_1_output, emo_cls_2_output,
      spk, emo)
    return tuple(out[0, k] for k in range(10))


def kernel(mels, pitches, energies, durations, speakers, emotions, output,
           postnet_output, p_preds, e_preds, d_preds, src_masks, mel_masks,
           spk_cls_1_output, spk_cls_2_output, emo_cls_1_output,
           emo_cls_2_output):
    return _run(mels, pitches, energies, durations, speakers, emotions,
                output, postnet_output, p_preds, e_preds, d_preds,
                src_masks, mel_masks, spk_cls_1_output, spk_cls_2_output,
                emo_cls_1_output, emo_cls_2_output)


# speedup vs baseline: 2.1088x; 2.0535x over previous
import jax
import jax.numpy as jnp
from jax import lax
from jax.experimental import pallas as pl
from jax.experimental.pallas import tpu as pltpu

B, S, T, M = 32, 200, 1000, 80
N_SPK, N_EMO = 256, 8

_DOT = (((1,), (0,)), ((), ()))


def _ce(logits, labels):
    mx = jnp.max(logits, axis=-1, keepdims=True)
    lse = mx + jnp.log(jnp.sum(jnp.exp(logits - mx), axis=-1, keepdims=True))
    logp = logits - lse
    onehot = lax.broadcasted_iota(jnp.int32, logits.shape, 1) == labels
    picked = jnp.sum(jnp.where(onehot, logp, 0.0))
    return -picked / float(B)


def _body(mel_ref, out_ref, post_ref, mv3_ref,
          mv_ref, pp_ref, pt_ref, ep_ref, et_ref,
          dp_ref, dt_ref, sv_ref,
          s1_ref, s2_ref, e1_ref, e2_ref, spk_ref, emo_ref,
          o_ref, acc_ref):
    i = pl.program_id(0)

    mv = mv3_ref[0]
    mel = mel_ref[0]
    dm = jnp.abs(out_ref[0] - mel)
    dpn = jnp.abs(post_ref[0] - mel)
    cm = lax.dot_general(mv, dm, _DOT, preferred_element_type=jnp.float32)
    cp = lax.dot_general(mv, dpn, _DOT, preferred_element_type=jnp.float32)

    @pl.when(i == 0)
    def _():
        acc_ref[0:1, :M] = cm
        acc_ref[1:2, :M] = cp

    @pl.when(i > 0)
    def _():
        acc_ref[0:1, :M] += cm
        acc_ref[1:2, :M] += cp

    @pl.when(i == B - 1)
    def _():
        mvd = mv_ref[...]
        s_mask = jnp.sum(mvd)
        denom3 = jnp.maximum(s_mask * float(M), 1.0)
        mel_loss = jnp.sum(acc_ref[0:1, :M]) / denom3
        postnet_mel_loss = jnp.sum(acc_ref[1:2, :M]) / denom3

        denom1 = jnp.maximum(s_mask, 1.0)
        pitch_loss = jnp.sum((pp_ref[...] - pt_ref[...]) ** 2 * mvd) / denom1
        energy_loss = jnp.sum((ep_ref[...] - et_ref[...]) ** 2 * mvd) / denom1

        sv = sv_ref[...]
        log_dur = jnp.log(dt_ref[...] + 1.0)
        duration_loss = (jnp.sum((dp_ref[...] - log_dur) ** 2 * sv)
                         / jnp.maximum(jnp.sum(sv), 1.0))

        spk = spk_ref[...]
        emo = emo_ref[...]
        speaker_loss_1 = _ce(s1_ref[...], spk)
        speaker_loss_2 = _ce(s2_ref[...], spk)
        emotion_loss_1 = _ce(e1_ref[...], emo)
        emotion_loss_2 = _ce(e2_ref[...], emo)

        all_loss = (mel_loss + postnet_mel_loss + pitch_loss + energy_loss
                    + duration_loss)
        total_loss = (all_loss + speaker_loss_1 + emotion_loss_1
                      + speaker_loss_2 + emotion_loss_2)

        vals = (mel_loss, postnet_mel_loss, pitch_loss, energy_loss,
                duration_loss, speaker_loss_1, speaker_loss_2,
                emotion_loss_1, emotion_loss_2, total_loss)
        col = lax.broadcasted_iota(jnp.int32, (8, 128), 1)
        row = lax.broadcasted_iota(jnp.int32, (8, 128), 0)
        res = jnp.zeros((8, 128), jnp.float32)
        for k, v in enumerate(vals):
            res = jnp.where((row == 0) & (col == k), v, res)
        o_ref[...] = res


@jax.jit
def _run(mels, pitches, energies, durations, speakers, emotions, output,
         postnet_output, p_preds, e_preds, d_preds, src_masks, mel_masks,
         spk_cls_1_output, spk_cls_2_output, emo_cls_1_output,
         emo_cls_2_output):
    mel_valid = (~mel_masks).astype(jnp.float32)
    src_valid = (~src_masks).astype(jnp.float32)
    mv3 = mel_valid.reshape(B, 1, T)

    dur_f = durations.astype(jnp.float32)
    spk = speakers.astype(jnp.int32).reshape(B, 1)
    emo = emotions.astype(jnp.int32).reshape(B, 1)

    big = pl.BlockSpec((1, T, M), lambda i: (i, 0, 0))
    whole = lambda r, c: pl.BlockSpec((r, c), lambda i: (0, 0))

    out = pl.pallas_call(
        _body,
        grid=(B,),
        in_specs=[
            big, big, big,
            pl.BlockSpec((1, 1, T), lambda i: (i, 0, 0)),
            whole(B, T), whole(B, T), whole(B, T),
            whole(B, T), whole(B, T),
            whole(B, S), whole(B, S), whole(B, S),
            whole(B, N_SPK), whole(B, N_SPK),
            whole(B, N_EMO), whole(B, N_EMO),
            whole(B, 1), whole(B, 1),
        ],
        out_specs=pl.BlockSpec((8, 128), lambda i: (0, 0)),
        out_shape=jax.ShapeDtypeStruct((8, 128), jnp.float32),
        scratch_shapes=[pltpu.VMEM((8, 128), jnp.float32)],
    )(mels, output, postnet_output, mv3,
      mel_valid, p_preds, pitches, e_preds, energies,
      d_preds, dur_f, src_valid,
      spk_cls_1_output, spk_cls_2_output,
      emo_cls_1_output, emo_cls_2_output,
      spk, emo)
    return tuple(out[0, k] for k in range(10))


def kernel(mels, pitches, energies, durations, speakers, emotions, output,
           postnet_output, p_preds, e_preds, d_preds, src_masks, mel_masks,
           spk_cls_1_output, spk_cls_2_output, emo_cls_1_output,
           emo_cls_2_output):
    return _run(mels, pitches, energies, durations, speakers, emotions,
                output, postnet_output, p_preds, e_preds, d_preds,
                src_masks, mel_masks, spk_cls_1_output, spk_cls_2_output,
                emo_cls_1_output, emo_cls_2_output)
